# baseline (speedup 1.0000x reference)
import jax
import jax.numpy as jnp
from jax.experimental import pallas as pl
from jax.experimental.pallas import tpu as pltpu

_DN = (((1,), (1,)), ((), ()))

_CM = 400
_NBUF = 3


def _sage_manual_kernel(
    adj_hbm, feat_ref, w_ref, b_ref, out_hbm, bufs, obufs, sems, osems
):
    n, d = feat_ref.shape
    nc = n // _CM

    def in_copy(i, slot):
        return pltpu.make_async_copy(
            adj_hbm.at[pl.ds(i * _CM, _CM), :], bufs.at[slot], sems.at[slot]
        )

    def out_copy(i, slot):
        return pltpu.make_async_copy(
            obufs.at[slot], out_hbm.at[pl.ds(i * _CM, _CM), :], osems.at[slot]
        )

    for s in range(_NBUF):
        in_copy(s, s).start()

    def loop(i, carry):
        slot = jax.lax.rem(i, _NBUF)
        in_copy(i, slot).wait()
        a = bufs[slot]
        nb = jnp.dot(a, feat_ref[...], preferred_element_type=jnp.float32)
        self_f = feat_ref[pl.ds(i * _CM, _CM), :]
        out = (
            jax.lax.dot_general(
                self_f, w_ref[:, 0:d], _DN, preferred_element_type=jnp.float32
            )
            + jax.lax.dot_general(
                nb, w_ref[:, d : 2 * d], _DN, preferred_element_type=jnp.float32
            )
            + b_ref[...]
        )
        norm = jnp.sqrt(jnp.sum(out * out, axis=1, keepdims=True))

        @pl.when(i >= _NBUF)
        def _():
            out_copy(i - _NBUF, slot).wait()

        obufs[slot] = out / jnp.maximum(norm, 1e-12)
        out_copy(i, slot).start()

        @pl.when(i + _NBUF < nc)
        def _():
            in_copy(i + _NBUF, slot).start()

        return carry

    jax.lax.fori_loop(0, nc, loop, 0)

    for s in range(_NBUF):
        i = nc - _NBUF + s
        out_copy(i, i % _NBUF).wait()


def kernel(features, adj, W, b):
    n, d = features.shape
    b2 = b.reshape(1, d)
    return pl.pallas_call(
        _sage_manual_kernel,
        in_specs=[
            pl.BlockSpec(memory_space=pl.ANY),
            pl.BlockSpec(memory_space=pltpu.MemorySpace.VMEM),
            pl.BlockSpec(memory_space=pltpu.MemorySpace.VMEM),
            pl.BlockSpec(memory_space=pltpu.MemorySpace.VMEM),
        ],
        out_specs=pl.BlockSpec(memory_space=pl.ANY),
        out_shape=jax.ShapeDtypeStruct((n, d), jnp.float32),
        scratch_shapes=[
            pltpu.VMEM((_NBUF, _CM, 10000), jnp.float32),
            pltpu.VMEM((_NBUF, _CM, 128), jnp.float32),
            pltpu.SemaphoreType.DMA((_NBUF,)),
            pltpu.SemaphoreType.DMA((_NBUF,)),
        ],
        compiler_params=pltpu.CompilerParams(
            vmem_limit_bytes=100 * 1024 * 1024,
        ),
    )(adj, features, W, b2)


# speedup vs baseline: 1.0315x; 1.0315x over previous
import jax
import jax.numpy as jnp
from jax.experimental import pallas as pl
from jax.experimental.pallas import tpu as pltpu

_DN = (((1,), (1,)), ((), ()))


def _sage_block_kernel(adj_ref, feat_ref, w_ref, b_ref, out_ref):
    i = pl.program_id(0)
    bm, d = out_ref.shape
    nb = jnp.dot(adj_ref[...], feat_ref[...], preferred_element_type=jnp.float32)
    self_f = feat_ref[pl.ds(i * bm, bm), :]
    out = (
        jax.lax.dot_general(
            self_f, w_ref[:, 0:d], _DN, preferred_element_type=jnp.float32
        )
        + jax.lax.dot_general(
            nb, w_ref[:, d : 2 * d], _DN, preferred_element_type=jnp.float32
        )
        + b_ref[...]
    )
    norm = jnp.sqrt(jnp.sum(out * out, axis=1, keepdims=True))
    out_ref[...] = out / jnp.maximum(norm, 1e-12)


def kernel(features, adj, W, b):
    n, d = features.shape
    bm = 400
    b2 = b.reshape(1, d)
    return pl.pallas_call(
        _sage_block_kernel,
        grid=(n // bm,),
        in_specs=[
            pl.BlockSpec((bm, n), lambda i: (i, 0)),
            pl.BlockSpec((n, d), lambda i: (0, 0)),
            pl.BlockSpec((d, 2 * d), lambda i: (0, 0)),
            pl.BlockSpec((1, d), lambda i: (0, 0)),
        ],
        out_specs=pl.BlockSpec((bm, d), lambda i: (i, 0)),
        out_shape=jax.ShapeDtypeStruct((n, d), jnp.float32),
        compiler_params=pltpu.CompilerParams(
            dimension_semantics=("arbitrary",),
            vmem_limit_bytes=100 * 1024 * 1024,
        ),
    )(adj, features, W, b2)
